# baseline (speedup 1.0000x reference)
import functools

import jax
import jax.numpy as jnp
from jax import lax
from jax.experimental import pallas as pl
from jax.experimental.pallas import tpu as pltpu
from jax.experimental.pallas import tpu_sc as plsc

N_USERS = 2000
N_ITEMS = 8000
NN = N_USERS + N_ITEMS
D = 256
NLAYER = 3
E = 160000
B = 4096

NC, NS, LN = 2, 16, 16
NW = NC * NS

NPT = 320
NNP = NW * NPT
DEG_R, DEG_C = 80, 128
NPADDED = DEG_R * DEG_C
E_PAD = 163840
EPW = E_PAD // NW
PAD_COL = 10100
CAP = 6144
ECH = 4096
GB = 64
BPW = B // NW
CB = 16

_f32 = jnp.float32
_i32 = jnp.int32

_sc_mesh = None


def _mesh():
    global _sc_mesh
    if _sc_mesh is None:
        _sc_mesh = plsc.VectorSubcoreMesh(
            core_axis_name="c", subcore_axis_name="s",
            num_cores=NC, num_subcores=NS,
        )
    return _sc_mesh


_SC_PARAMS = pltpu.CompilerParams(needs_layout_passes=False)
_SC_PARAMS_NOTC = pltpu.CompilerParams(
    needs_layout_passes=False, use_tc_tiling_on_sc=False)


def _wid():
    return lax.axis_index("c") * NS + lax.axis_index("s")


def _deg_body(col_hbm, out_hbm, deg_v, col_v, rid_v, deg_sh):
    s = lax.axis_index("s")
    c = lax.axis_index("c")
    wid = c * NS + s

    zeros = jnp.zeros((LN,), _f32)
    ones = jnp.ones((LN,), _f32)

    def zbody(r, carry):
        for k in range(DEG_C // LN):
            deg_v[r, pl.ds(k * LN, LN)] = zeros
        return carry

    lax.fori_loop(_i32(0), _i32(DEG_R), zbody, _i32(0))

    base_iota = lax.iota(_i32, LN)
    for i in range(DEG_R // LN):
        rid_v[pl.ds(i * LN, LN)] = base_iota + _i32(i * LN)

    pltpu.sync_copy(col_hbm.at[pl.ds(wid * _i32(EPW), EPW)], col_v)

    def body(i, carry):
        cv = col_v[pl.ds(i * _i32(LN), LN)]
        r = lax.shift_right_logical(cv, _i32(7))
        cc = lax.bitwise_and(cv, _i32(127))
        plsc.addupdate_scatter(deg_v, [r, cc], ones)
        return carry

    lax.fori_loop(_i32(0), _i32(EPW // LN), body, _i32(0))

    @pl.when(s == 0)
    def _():
        pltpu.sync_copy(deg_v, deg_sh)

    plsc.subcore_barrier()

    @pl.when(s != 0)
    def _():
        pltpu.sync_copy(deg_v, deg_sh.at[rid_v], add=True)

    plsc.subcore_barrier()

    @pl.when(s == 0)
    def _():
        pltpu.sync_copy(deg_sh, out_hbm.at[c])


def _deg_call(col_p):
    return pl.kernel(
        _deg_body,
        out_type=jax.ShapeDtypeStruct((NC, DEG_R, DEG_C), _f32),
        mesh=_mesh(),
        scratch_types=[
            pltpu.VMEM((DEG_R, DEG_C), _f32),
            pltpu.VMEM((EPW,), _i32),
            pltpu.VMEM((DEG_R,), _i32),
            pltpu.VMEM_SHARED((DEG_R, DEG_C), _f32),
        ],
        compiler_params=_SC_PARAMS,
    )(col_p)


def _dis_body(deg_ref, dis_ref):
    d = deg_ref[0] + deg_ref[1]
    dis_ref[...] = jnp.where(d > 0, lax.rsqrt(d), 0.0).astype(_f32)


def _dis_call(deg2):
    return pl.pallas_call(
        _dis_body,
        out_shape=jax.ShapeDtypeStruct((DEG_R, DEG_C), _f32),
    )(deg2)


def _filter_body(row_hbm, col_hbm, dis_hbm,
                 rows_hbm, cols_hbm, nrm_hbm, cnt_hbm,
                 dis_v, er_v, ec_v, rows_o, cols_o, nrm_o, cnt_o):
    wid = _wid()
    lo = wid * _i32(NPT)
    lo_v = jnp.full((LN,), lo, _i32)
    hi_v = lo_v + _i32(NPT)

    pltpu.sync_copy(dis_hbm, dis_v)

    iota = lax.iota(_i32, LN)
    zf = jnp.zeros((LN,), _f32)
    zi = jnp.zeros((LN,), _i32)
    onei = jnp.ones((LN,), _i32)

    def ibody(k, carry):
        sl = pl.ds(k * _i32(LN), LN)
        rows_o[sl] = lax.bitwise_and(iota + k * _i32(LN), _i32(8191))
        cols_o[sl] = zi
        nrm_o[sl] = zf
        return carry

    lax.fori_loop(_i32(0), _i32(CAP // LN), ibody, _i32(0))

    def chunk_body(ci, off_v):
        base = ci * _i32(ECH)
        pltpu.sync_copy(row_hbm.at[pl.ds(base, ECH)], er_v)
        pltpu.sync_copy(col_hbm.at[pl.ds(base, ECH)], ec_v)

        def body(i, off_v):
            sl = pl.ds(i * _i32(LN), LN)
            cv = ec_v[sl]
            rv = er_v[sl]
            m = jnp.logical_and(jnp.logical_and(cv >= lo_v, cv < hi_v),
                                cv < jnp.full((LN,), NN, _i32))
            dr = plsc.load_gather(dis_v, [rv])
            dc = plsc.load_gather(dis_v, [cv])
            nv = dr * dc
            mi = jnp.where(m, onei, zi)
            pc = plsc.cumsum(mi)
            idx = off_v + pc - onei
            plsc.store_scatter(rows_o, [idx], rv, mask=m)
            plsc.store_scatter(cols_o, [idx], cv - lo_v, mask=m)
            plsc.store_scatter(nrm_o, [idx], nv, mask=m)
            cnt = plsc.all_reduce_population_count(m)
            if cnt.shape != (LN,):
                cnt = jnp.broadcast_to(cnt, (LN,)).astype(_i32)
            return off_v + cnt

        return lax.fori_loop(_i32(0), _i32(ECH // LN), body, off_v)

    off_v = lax.fori_loop(_i32(0), _i32(E_PAD // ECH), chunk_body, zi)
    cnt_o[...] = off_v

    pltpu.sync_copy(rows_o, rows_hbm.at[wid, _i32(0)])
    pltpu.sync_copy(cols_o, cols_hbm.at[wid, _i32(0)])
    pltpu.sync_copy(nrm_o, nrm_hbm.at[wid, _i32(0)])
    pltpu.sync_copy(cnt_o, cnt_hbm.at[wid, _i32(0)])


def _filter_call(row_p, col_p, dis_flat):
    return pl.kernel(
        _filter_body,
        out_type=(
            jax.ShapeDtypeStruct((NW, 1, CAP), _i32),
            jax.ShapeDtypeStruct((NW, 1, CAP), _i32),
            jax.ShapeDtypeStruct((NW, 1, CAP), _f32),
            jax.ShapeDtypeStruct((NW, 1, LN), _i32),
        ),
        mesh=_mesh(),
        scratch_types=[
            pltpu.VMEM((NPADDED,), _f32),
            pltpu.VMEM((ECH,), _i32),
            pltpu.VMEM((ECH,), _i32),
            pltpu.VMEM((CAP,), _i32),
            pltpu.VMEM((CAP,), _i32),
            pltpu.VMEM((CAP,), _f32),
            pltpu.VMEM((LN,), _i32),
        ],
        compiler_params=_SC_PARAMS,
    )(row_p, col_p, dis_flat)


DH = 128
DHW = DH // 2
GB2 = 64
NUNIT = 2
WCH = 32


def _gather_batch(x_sh, rows_v, base, gb, sem):
    return pltpu.async_copy(x_sh.at[rows_v.at[pl.ds(base, GB2)]], gb, sem)


def _gather_batch_hbm(x_hbm, c, rows_v, base, gb, sem):
    return pltpu.async_copy(
        x_hbm.at[c].at[rows_v.at[pl.ds(base, GB2)]], gb, sem)


def _drain(x_hbm, c, gb, sem):
    pltpu.make_async_copy(x_hbm.at[c, pl.ds(_i32(0), GB2)], gb, sem).wait()


def _edge_compute(gb, acc, cols_v, nrm_v, base, iota, onei):
    for g in range(GB2 // LN):
        sl = pl.ds(base + _i32(g * LN), LN)
        colv = cols_v[sl]
        nrmv = nrm_v[sl]
        ev = iota + _i32(g * LN)

        def jbody(j, carry):
            oidx, jv = carry
            for u in range(8):
                w = plsc.load_gather(gb, [ev, jv])
                lo = plsc.bitcast(lax.shift_left(w, _i32(16)), _f32)
                hi = plsc.bitcast(lax.bitwise_and(w, _i32(-65536)), _f32)
                plsc.addupdate_scatter(acc, [oidx], lo * nrmv)
                plsc.addupdate_scatter(acc, [oidx + _i32(DHW)], hi * nrmv)
                oidx = oidx + onei
                jv = jv + onei
            return (oidx, jv)

        lax.fori_loop(_i32(0), _i32(DHW // 8), jbody,
                      (colv * _i32(DH), jnp.zeros((LN,), _i32)))


def _layer_body(x_hbm, rows_hbm, cols_hbm, nrm_hbm, cnt_hbm, y_hbm,
                x_sh, acc, gb0, gb1, gb2, gb3, ybuf,
                rows_v, cols_v, nrm_v, cnt_v,
                sem0, sem1, sem2, sem3):
    c = lax.axis_index("c")
    s = lax.axis_index("s")

    stripe = NNP // NS
    st = s * _i32(stripe)
    pltpu.sync_copy(x_hbm.at[c].at[pl.ds(st, stripe)],
                    x_sh.at[pl.ds(st, stripe)])
    iota = lax.iota(_i32, LN)
    zeros = jnp.zeros((LN,), _f32)
    onei = jnp.ones((LN,), _i32)
    plsc.subcore_barrier()

    for ui in range(NUNIT):
        u = s + _i32(ui * NS)
        pltpu.sync_copy(rows_hbm.at[u, _i32(0)], rows_v)
        pltpu.sync_copy(cols_hbm.at[u, _i32(0)], cols_v)
        pltpu.sync_copy(nrm_hbm.at[u, _i32(0)], nrm_v)
        pltpu.sync_copy(cnt_hbm.at[u, _i32(0)], cnt_v)

        def zbody(r, carry):
            acc[pl.ds(r * _i32(LN), LN)] = zeros
            return carry

        lax.fori_loop(_i32(0), _i32(NPT * DH // LN), zbody, _i32(0))

        n = jnp.max(cnt_v[...])
        npq = lax.div(n + _i32(4 * GB2 - 1), _i32(4 * GB2))

        gbs = (gb0, gb1, gb2, gb3)
        sems = (sem0, sem1, sem2, sem3)
        for q in range(3):
            _gather_batch(x_sh, rows_v, _i32(q * GB2), gbs[q], sems[q])

        def pbody(p, carry):
            for q in range(4):
                bi = (p * _i32(4) + _i32(q + 3)) * _i32(GB2)
                _gather_batch(x_sh, rows_v, bi, gbs[(q + 3) % 4],
                              sems[(q + 3) % 4])
                _drain(x_hbm, c, gbs[q], sems[q])
                _edge_compute(gbs[q], acc, cols_v, nrm_v,
                              (p * _i32(4) + _i32(q)) * _i32(GB2), iota, onei)
            return carry

        lax.fori_loop(_i32(0), npq, pbody, _i32(0))
        for q in range(3):
            _drain(x_hbm, c, gbs[q], sems[q])

        def wbody(rb, carry):
            def rbody(rr, carry2):
                rbase = (rb * _i32(WCH) + rr) * _i32(DH)
                for k in range(DHW // LN):
                    a = acc[pl.ds(rbase + _i32(k * LN), LN)]
                    b = acc[pl.ds(rbase + _i32(DHW + k * LN), LN)]
                    pk = plsc.pack(a, b, format=plsc.PackFormat.INTERLEAVED)
                    ybuf[rr, pl.ds(k * LN, LN)] = plsc.bitcast(pk, _i32)
                return carry2

            lax.fori_loop(_i32(0), _i32(WCH), rbody, _i32(0))
            pltpu.sync_copy(
                ybuf, y_hbm.at[c, pl.ds(u * _i32(NPT) + rb * _i32(WCH), WCH)])
            return carry

        lax.fori_loop(_i32(0), _i32(NPT // WCH), wbody, _i32(0))


def _layer_call(x, rows, cols, nrm, cnt):
    return pl.kernel(
        _layer_body,
        out_type=jax.ShapeDtypeStruct((NC, NNP, DHW), _i32),
        mesh=_mesh(),
        scratch_types=[
            pltpu.VMEM_SHARED((NNP, DHW), _i32),
            pltpu.VMEM((NPT * DH,), _f32),
            pltpu.VMEM((GB2, DHW), _i32),
            pltpu.VMEM((GB2, DHW), _i32),
            pltpu.VMEM((GB2, DHW), _i32),
            pltpu.VMEM((GB2, DHW), _i32),
            pltpu.VMEM((WCH, DHW), _i32),
            pltpu.VMEM((CAP,), _i32),
            pltpu.VMEM((CAP,), _i32),
            pltpu.VMEM((CAP,), _f32),
            pltpu.VMEM((LN,), _i32),
            pltpu.SemaphoreType.DMA,
            pltpu.SemaphoreType.DMA,
            pltpu.SemaphoreType.DMA,
            pltpu.SemaphoreType.DMA,
        ],
        compiler_params=_SC_PARAMS_NOTC,
    )(x, rows, cols, nrm, cnt)


def _unpack_tc(w):
    lo = lax.bitcast_convert_type(lax.shift_left(w, _i32(16)), _f32)
    hi = lax.bitcast_convert_type(lax.bitwise_and(w, _i32(-65536)), _f32)
    return lo, hi


def _xsum_body(a_ref, b_ref, c_ref, d_ref, o_ref):
    alo, ahi = _unpack_tc(a_ref[...])
    blo, bhi = _unpack_tc(b_ref[...])
    clo, chi = _unpack_tc(c_ref[...])
    dlo, dhi = _unpack_tc(d_ref[...])
    o_ref[:, :, :DHW] = 0.25 * (alo + blo + clo + dlo)
    o_ref[:, :, DHW:] = 0.25 * (ahi + bhi + chi + dhi)


def _xsum_call(x0, x1, x2, x3):
    iblk = pl.BlockSpec((1, 2048, DHW), lambda i, j: (i, j, _i32(0)))
    oblk = pl.BlockSpec((1, 2048, DH), lambda i, j: (i, j, _i32(0)))
    return pl.pallas_call(
        _xsum_body,
        out_shape=jax.ShapeDtypeStruct((NC, NNP, DH), _f32),
        grid=(NC, NNP // 2048),
        in_specs=[iblk, iblk, iblk, iblk],
        out_specs=oblk,
    )(x0, x1, x2, x3)


BPW2 = B // NS


def _score_body(xs_hbm, uix_hbm, pix_hbm, nix_hbm,
                posp_hbm, negp_hbm,
                x_sh, uix_v, pix_v, nix_v, ub, pb, nb_, pos_o, neg_o, sem):
    c = lax.axis_index("c")
    s = lax.axis_index("s")

    @pl.when(s == 0)
    def _():
        pltpu.sync_copy(xs_hbm.at[c], x_sh)

    base = s * _i32(BPW2)
    pltpu.sync_copy(uix_hbm.at[pl.ds(base, BPW2)], uix_v)
    pltpu.sync_copy(pix_hbm.at[pl.ds(base, BPW2)], pix_v)
    pltpu.sync_copy(nix_hbm.at[pl.ds(base, BPW2)], nix_v)
    plsc.subcore_barrier()

    iota = lax.iota(_i32, LN)
    zf = jnp.zeros((LN,), _f32)

    for k in range(BPW2 // CB):
        ksl = pl.ds(_i32(k * CB), CB)
        pltpu.async_copy(x_sh.at[uix_v.at[ksl]], ub, sem)
        pltpu.async_copy(x_sh.at[pix_v.at[ksl]], pb, sem)
        pltpu.async_copy(x_sh.at[nix_v.at[ksl]], nb_, sem).wait()
        pltpu.make_async_copy(xs_hbm.at[c, pl.ds(_i32(0), CB)], ub, sem).wait()
        pltpu.make_async_copy(xs_hbm.at[c, pl.ds(_i32(0), CB)], pb, sem).wait()

        def jbody(j, carry):
            accp, accn = carry
            for u in range(4):
                jv = jnp.full((LN,), j * _i32(4) + _i32(u), _i32)
                uv = plsc.load_gather(ub, [iota, jv])
                pv = plsc.load_gather(pb, [iota, jv])
                nv = plsc.load_gather(nb_, [iota, jv])
                accp = accp + uv * pv
                accn = accn + uv * nv
            return (accp, accn)

        accp, accn = lax.fori_loop(_i32(0), _i32(DH // 4), jbody, (zf, zf))
        pos_o[pl.ds(k * CB, CB)] = accp
        neg_o[pl.ds(k * CB, CB)] = accn

    pltpu.sync_copy(pos_o, posp_hbm.at[c, pl.ds(base, BPW2)])
    pltpu.sync_copy(neg_o, negp_hbm.at[c, pl.ds(base, BPW2)])


def _score_call(xsum, uix, pix, nix):
    return pl.kernel(
        _score_body,
        out_type=(
            jax.ShapeDtypeStruct((NC, B), _f32),
            jax.ShapeDtypeStruct((NC, B), _f32),
        ),
        mesh=_mesh(),
        scratch_types=[
            pltpu.VMEM_SHARED((NNP, DH), _f32),
            pltpu.VMEM((BPW2,), _i32),
            pltpu.VMEM((BPW2,), _i32),
            pltpu.VMEM((BPW2,), _i32),
            pltpu.VMEM((CB, DH), _f32),
            pltpu.VMEM((CB, DH), _f32),
            pltpu.VMEM((CB, DH), _f32),
            pltpu.VMEM((BPW2,), _f32),
            pltpu.VMEM((BPW2,), _f32),
            pltpu.SemaphoreType.DMA,
        ],
        compiler_params=_SC_PARAMS,
    )(xsum, uix, pix, nix)


def _comb_body(pp_ref, np_ref, pos_ref, neg_ref):
    pos_ref[...] = pp_ref[0] + pp_ref[1]
    neg_ref[...] = np_ref[0] + np_ref[1]


def _comb_call(posp, negp):
    return pl.pallas_call(
        _comb_body,
        out_shape=(
            jax.ShapeDtypeStruct((B,), _f32),
            jax.ShapeDtypeStruct((B,), _f32),
        ),
    )(posp, negp)


def kernel(edge_index, user, pos_item, neg_item, user_w, item_w):
    row = edge_index[0].astype(_i32)
    col = edge_index[1].astype(_i32)
    pad = E_PAD - E
    row_p = jnp.concatenate([row, jnp.arange(pad, dtype=_i32) % NN])
    col_p = jnp.concatenate([col, jnp.full((pad,), PAD_COL, _i32)])

    deg2 = _deg_call(col_p)
    dis = _dis_call(deg2).reshape(NPADDED)
    rows, cols, nrm, cnt = _filter_call(row_p, col_p, dis)

    xf = jnp.concatenate(
        [user_w, item_w, jnp.zeros((NNP - NN, D), _f32)], axis=0)

    def _pack_half(xh):
        lo = lax.bitcast_convert_type(
            xh[:, :DHW].astype(jnp.bfloat16), jnp.uint16).astype(jnp.uint32)
        hi = lax.bitcast_convert_type(
            xh[:, DHW:].astype(jnp.bfloat16), jnp.uint16).astype(jnp.uint32)
        return lax.bitcast_convert_type(lo | (hi << 16), _i32)

    x0 = jnp.stack([_pack_half(xf[:, :DH]), _pack_half(xf[:, DH:])], axis=0)
    x1 = _layer_call(x0, rows, cols, nrm, cnt)
    x2 = _layer_call(x1, rows, cols, nrm, cnt)
    x3 = _layer_call(x2, rows, cols, nrm, cnt)
    xsum = _xsum_call(x0, x1, x2, x3)

    uix = user.astype(_i32)
    pix = N_USERS + pos_item.astype(_i32)
    nix = N_USERS + neg_item.astype(_i32)
    posp, negp = _score_call(xsum, uix, pix, nix)
    return _comb_call(posp, negp)


# speedup vs baseline: 1.0555x; 1.0555x over previous
import functools

import jax
import jax.numpy as jnp
from jax import lax
from jax.experimental import pallas as pl
from jax.experimental.pallas import tpu as pltpu
from jax.experimental.pallas import tpu_sc as plsc

N_USERS = 2000
N_ITEMS = 8000
NN = N_USERS + N_ITEMS
D = 256
NLAYER = 3
E = 160000
B = 4096

NC, NS, LN = 2, 16, 16
NW = NC * NS

NPT = 320
NNP = NW * NPT
DEG_R, DEG_C = 80, 128
NPADDED = DEG_R * DEG_C
E_PAD = 163840
EPW = E_PAD // NW
PAD_COL = 10100
CAP = 6144
ECH = 4096
GB = 64
BPW = B // NW
CB = 16

_f32 = jnp.float32
_i32 = jnp.int32

_sc_mesh = None


def _mesh():
    global _sc_mesh
    if _sc_mesh is None:
        _sc_mesh = plsc.VectorSubcoreMesh(
            core_axis_name="c", subcore_axis_name="s",
            num_cores=NC, num_subcores=NS,
        )
    return _sc_mesh


_SC_PARAMS = pltpu.CompilerParams(needs_layout_passes=False)
_SC_PARAMS_NOTC = pltpu.CompilerParams(
    needs_layout_passes=False, use_tc_tiling_on_sc=False)


def _wid():
    return lax.axis_index("c") * NS + lax.axis_index("s")


def _deg_body(col_hbm, out_hbm, deg_v, col_v, rid_v, deg_sh):
    s = lax.axis_index("s")
    c = lax.axis_index("c")
    wid = c * NS + s

    zeros = jnp.zeros((LN,), _f32)
    ones = jnp.ones((LN,), _f32)

    def zbody(r, carry):
        for k in range(DEG_C // LN):
            deg_v[r, pl.ds(k * LN, LN)] = zeros
        return carry

    lax.fori_loop(_i32(0), _i32(DEG_R), zbody, _i32(0))

    base_iota = lax.iota(_i32, LN)
    for i in range(DEG_R // LN):
        rid_v[pl.ds(i * LN, LN)] = base_iota + _i32(i * LN)

    pltpu.sync_copy(col_hbm.at[pl.ds(wid * _i32(EPW), EPW)], col_v)

    def body(i, carry):
        cv = col_v[pl.ds(i * _i32(LN), LN)]
        r = lax.shift_right_logical(cv, _i32(7))
        cc = lax.bitwise_and(cv, _i32(127))
        plsc.addupdate_scatter(deg_v, [r, cc], ones)
        return carry

    lax.fori_loop(_i32(0), _i32(EPW // LN), body, _i32(0))

    @pl.when(s == 0)
    def _():
        pltpu.sync_copy(deg_v, deg_sh)

    plsc.subcore_barrier()

    @pl.when(s != 0)
    def _():
        pltpu.sync_copy(deg_v, deg_sh.at[rid_v], add=True)

    plsc.subcore_barrier()

    @pl.when(s == 0)
    def _():
        pltpu.sync_copy(deg_sh, out_hbm.at[c])


def _deg_call(col_p):
    return pl.kernel(
        _deg_body,
        out_type=jax.ShapeDtypeStruct((NC, DEG_R, DEG_C), _f32),
        mesh=_mesh(),
        scratch_types=[
            pltpu.VMEM((DEG_R, DEG_C), _f32),
            pltpu.VMEM((EPW,), _i32),
            pltpu.VMEM((DEG_R,), _i32),
            pltpu.VMEM_SHARED((DEG_R, DEG_C), _f32),
        ],
        compiler_params=_SC_PARAMS,
    )(col_p)


def _dis_body(deg_ref, dis_ref):
    d = deg_ref[0] + deg_ref[1]
    dis_ref[...] = jnp.where(d > 0, lax.rsqrt(d), 0.0).astype(_f32)


def _dis_call(deg2):
    return pl.pallas_call(
        _dis_body,
        out_shape=jax.ShapeDtypeStruct((DEG_R, DEG_C), _f32),
    )(deg2)


def _filter_body(row_hbm, col_hbm, dis_hbm,
                 rows_hbm, cols_hbm, nrm_hbm, cnt_hbm,
                 dis_v, er_v, ec_v, rows_o, cols_o, nrm_o, cnt_o):
    wid = _wid()
    lo = wid * _i32(NPT)
    lo_v = jnp.full((LN,), lo, _i32)
    hi_v = lo_v + _i32(NPT)

    pltpu.sync_copy(dis_hbm, dis_v)

    iota = lax.iota(_i32, LN)
    zf = jnp.zeros((LN,), _f32)
    zi = jnp.zeros((LN,), _i32)
    onei = jnp.ones((LN,), _i32)

    def ibody(k, carry):
        sl = pl.ds(k * _i32(LN), LN)
        rows_o[sl] = lax.bitwise_and(iota + k * _i32(LN), _i32(8191))
        cols_o[sl] = zi
        nrm_o[sl] = zf
        return carry

    lax.fori_loop(_i32(0), _i32(CAP // LN), ibody, _i32(0))

    def chunk_body(ci, off_v):
        base = ci * _i32(ECH)
        pltpu.sync_copy(row_hbm.at[pl.ds(base, ECH)], er_v)
        pltpu.sync_copy(col_hbm.at[pl.ds(base, ECH)], ec_v)

        def body(i, off_v):
            sl = pl.ds(i * _i32(LN), LN)
            cv = ec_v[sl]
            rv = er_v[sl]
            m = jnp.logical_and(jnp.logical_and(cv >= lo_v, cv < hi_v),
                                cv < jnp.full((LN,), NN, _i32))
            dr = plsc.load_gather(dis_v, [rv])
            dc = plsc.load_gather(dis_v, [cv])
            nv = dr * dc
            mi = jnp.where(m, onei, zi)
            pc = plsc.cumsum(mi)
            idx = off_v + pc - onei
            plsc.store_scatter(rows_o, [idx], rv, mask=m)
            plsc.store_scatter(cols_o, [idx], cv - lo_v, mask=m)
            plsc.store_scatter(nrm_o, [idx], nv, mask=m)
            cnt = plsc.all_reduce_population_count(m)
            if cnt.shape != (LN,):
                cnt = jnp.broadcast_to(cnt, (LN,)).astype(_i32)
            return off_v + cnt

        return lax.fori_loop(_i32(0), _i32(ECH // LN), body, off_v)

    off_v = lax.fori_loop(_i32(0), _i32(E_PAD // ECH), chunk_body, zi)
    cnt_o[...] = off_v

    pltpu.sync_copy(rows_o, rows_hbm.at[wid, _i32(0)])
    pltpu.sync_copy(cols_o, cols_hbm.at[wid, _i32(0)])
    pltpu.sync_copy(nrm_o, nrm_hbm.at[wid, _i32(0)])
    pltpu.sync_copy(cnt_o, cnt_hbm.at[wid, _i32(0)])


def _filter_call(row_p, col_p, dis_flat):
    return pl.kernel(
        _filter_body,
        out_type=(
            jax.ShapeDtypeStruct((NW, 1, CAP), _i32),
            jax.ShapeDtypeStruct((NW, 1, CAP), _i32),
            jax.ShapeDtypeStruct((NW, 1, CAP), _f32),
            jax.ShapeDtypeStruct((NW, 1, LN), _i32),
        ),
        mesh=_mesh(),
        scratch_types=[
            pltpu.VMEM((NPADDED,), _f32),
            pltpu.VMEM((ECH,), _i32),
            pltpu.VMEM((ECH,), _i32),
            pltpu.VMEM((CAP,), _i32),
            pltpu.VMEM((CAP,), _i32),
            pltpu.VMEM((CAP,), _f32),
            pltpu.VMEM((LN,), _i32),
        ],
        compiler_params=_SC_PARAMS,
    )(row_p, col_p, dis_flat)


DH = 128
DHW = DH // 2
GB2 = 64
NUNIT = 2
WCH = 32


def _gather_batch(x_sh, rows_v, base, gb, sem):
    return pltpu.async_copy(x_sh.at[rows_v.at[pl.ds(base, GB2)]], gb, sem)


def _gather_batch_hbm(x_hbm, c, rows_v, base, gb, sem):
    return pltpu.async_copy(
        x_hbm.at[c].at[rows_v.at[pl.ds(base, GB2)]], gb, sem)


def _drain(x_hbm, c, gb, sem):
    pltpu.make_async_copy(x_hbm.at[c, pl.ds(_i32(0), GB2)], gb, sem).wait()


def _edge_compute(gb, msg, nrm_v, base, iota, onei):
    for g in range(GB2 // LN):
        nrmv = nrm_v[pl.ds(base + _i32(g * LN), LN)]
        ev = iota + _i32(g * LN)

        def jbody(j, jv):
            for u in range(8):
                w = plsc.load_gather(gb, [ev, jv])
                lo = plsc.bitcast(lax.shift_left(w, _i32(16)), _f32)
                hi = plsc.bitcast(lax.bitwise_and(w, _i32(-65536)), _f32)
                plsc.store_scatter(msg, [ev, jv], lo * nrmv)
                plsc.store_scatter(msg, [ev, jv + _i32(DHW)], hi * nrmv)
                jv = jv + onei
            return jv

        lax.fori_loop(_i32(0), _i32(DHW // 8), jbody, jnp.zeros((LN,), _i32))


def _layer_body(x_hbm, rows_hbm, cols_hbm, nrm_hbm, cnt_hbm, dz_hbm, y_hbm,
                x_sh, acc_sh, gb0, gb1, gb2, msg, accbuf, ybuf,
                rows_v, cols_v, nrm_v, oix, cnt_v,
                sem0, sem1, sem2, ssem):
    c = lax.axis_index("c")
    s = lax.axis_index("s")
    stripe = NNP // NS
    st = s * _i32(stripe)
    pltpu.sync_copy(x_hbm.at[c].at[pl.ds(st, stripe)],
                    x_sh.at[pl.ds(st, stripe)])
    iota = lax.iota(_i32, LN)
    zeros = jnp.zeros((LN,), _f32)
    onei = jnp.ones((LN,), _i32)
    soff = s * _i32(NPT)
    plsc.subcore_barrier()

    gbs = (gb0, gb1, gb2)
    sems = (sem0, sem1, sem2)

    for ui in range(NUNIT):
        u = s + _i32(ui * NS)
        pltpu.sync_copy(rows_hbm.at[u, _i32(0)], rows_v)
        pltpu.sync_copy(cols_hbm.at[u, _i32(0)], cols_v)
        pltpu.sync_copy(nrm_hbm.at[u, _i32(0)], nrm_v)
        pltpu.sync_copy(cnt_hbm.at[u, _i32(0)], cnt_v)

        def abody(r, carry):
            accbuf[r, pl.ds(0, LN)] = zeros
            for k in range(1, DH // LN):
                accbuf[r, pl.ds(k * LN, LN)] = zeros
            return carry

        lax.fori_loop(_i32(0), _i32(WCH), abody, _i32(0))
        for k in range(NPT // WCH):
            pltpu.sync_copy(
                accbuf, acc_sh.at[pl.ds(soff + _i32(k * WCH), WCH)])

        n = jnp.max(cnt_v[...])
        ntr = lax.div(n + _i32(3 * GB2 - 1), _i32(3 * GB2))

        for q in range(3):
            _gather_batch(x_sh, rows_v, _i32(q * GB2), gbs[q], sems[q])

        def pbody(p, carry):
            for q in range(3):
                b = p * _i32(3) + _i32(q)
                bb = b * _i32(GB2)
                @pl.when(jnp.logical_or(p > 0, _i32(q) > 0))
                def _():
                    pltpu.make_async_copy(dz_hbm, msg, ssem).wait()

                _drain(x_hbm, c, gbs[q], sems[q])
                _edge_compute(gbs[q], msg, nrm_v, bb, iota, onei)
                for g in range(GB2 // LN):
                    oix[pl.ds(g * LN, LN)] = (
                        cols_v[pl.ds(bb + _i32(g * LN), LN)] + soff)
                pltpu.async_copy(msg, acc_sh.at[oix], ssem, add=True)
                _gather_batch(x_sh, rows_v, bb + _i32(3 * GB2), gbs[q],
                              sems[q])
            return carry

        lax.fori_loop(_i32(0), ntr, pbody, _i32(0))
        for q in range(3):
            _drain(x_hbm, c, gbs[q], sems[q])
        pltpu.make_async_copy(dz_hbm, msg, ssem).wait()

        def wbody(rb, carry):
            pltpu.sync_copy(
                acc_sh.at[pl.ds(soff + rb * _i32(WCH), WCH)], accbuf)

            def rbody(rr, carry2):
                for k in range(DHW // LN):
                    a = accbuf[rr, pl.ds(k * LN, LN)]
                    b = accbuf[rr, pl.ds(DHW + k * LN, LN)]
                    pk = plsc.pack(a, b, format=plsc.PackFormat.INTERLEAVED)
                    ybuf[rr, pl.ds(k * LN, LN)] = plsc.bitcast(pk, _i32)
                return carry2

            lax.fori_loop(_i32(0), _i32(WCH), rbody, _i32(0))
            pltpu.sync_copy(
                ybuf, y_hbm.at[c, pl.ds(u * _i32(NPT) + rb * _i32(WCH), WCH)])
            return carry

        lax.fori_loop(_i32(0), _i32(NPT // WCH), wbody, _i32(0))


def _layer_call(x, rows, cols, nrm, cnt, dz):
    return pl.kernel(
        _layer_body,
        out_type=jax.ShapeDtypeStruct((NC, NNP, DHW), _i32),
        mesh=_mesh(),
        scratch_types=[
            pltpu.VMEM_SHARED((NNP, DHW), _i32),
            pltpu.VMEM_SHARED((NS * NPT, DH), _f32),
            pltpu.VMEM((GB2, DHW), _i32),
            pltpu.VMEM((GB2, DHW), _i32),
            pltpu.VMEM((GB2, DHW), _i32),
            pltpu.VMEM((GB2, DH), _f32),
            pltpu.VMEM((WCH, DH), _f32),
            pltpu.VMEM((WCH, DHW), _i32),
            pltpu.VMEM((CAP,), _i32),
            pltpu.VMEM((CAP,), _i32),
            pltpu.VMEM((CAP,), _f32),
            pltpu.VMEM((GB2,), _i32),
            pltpu.VMEM((LN,), _i32),
            pltpu.SemaphoreType.DMA,
            pltpu.SemaphoreType.DMA,
            pltpu.SemaphoreType.DMA,
            pltpu.SemaphoreType.DMA,
        ],
        compiler_params=_SC_PARAMS_NOTC,
    )(x, rows, cols, nrm, cnt, dz)


def _unpack_tc(w):
    lo = lax.bitcast_convert_type(lax.shift_left(w, _i32(16)), _f32)
    hi = lax.bitcast_convert_type(lax.bitwise_and(w, _i32(-65536)), _f32)
    return lo, hi


def _xsum_body(a_ref, b_ref, c_ref, d_ref, o_ref):
    alo, ahi = _unpack_tc(a_ref[...])
    blo, bhi = _unpack_tc(b_ref[...])
    clo, chi = _unpack_tc(c_ref[...])
    dlo, dhi = _unpack_tc(d_ref[...])
    o_ref[:, :, :DHW] = 0.25 * (alo + blo + clo + dlo)
    o_ref[:, :, DHW:] = 0.25 * (ahi + bhi + chi + dhi)


def _xsum_call(x0, x1, x2, x3):
    iblk = pl.BlockSpec((1, 2048, DHW), lambda i, j: (i, j, _i32(0)))
    oblk = pl.BlockSpec((1, 2048, DH), lambda i, j: (i, j, _i32(0)))
    return pl.pallas_call(
        _xsum_body,
        out_shape=jax.ShapeDtypeStruct((NC, NNP, DH), _f32),
        grid=(NC, NNP // 2048),
        in_specs=[iblk, iblk, iblk, iblk],
        out_specs=oblk,
    )(x0, x1, x2, x3)


BPW2 = B // NS


def _score_body(xs_hbm, uix_hbm, pix_hbm, nix_hbm,
                posp_hbm, negp_hbm,
                x_sh, uix_v, pix_v, nix_v, ub, pb, nb_, pos_o, neg_o, sem):
    c = lax.axis_index("c")
    s = lax.axis_index("s")

    @pl.when(s == 0)
    def _():
        pltpu.sync_copy(xs_hbm.at[c], x_sh)

    base = s * _i32(BPW2)
    pltpu.sync_copy(uix_hbm.at[pl.ds(base, BPW2)], uix_v)
    pltpu.sync_copy(pix_hbm.at[pl.ds(base, BPW2)], pix_v)
    pltpu.sync_copy(nix_hbm.at[pl.ds(base, BPW2)], nix_v)
    plsc.subcore_barrier()

    iota = lax.iota(_i32, LN)
    zf = jnp.zeros((LN,), _f32)

    for k in range(BPW2 // CB):
        ksl = pl.ds(_i32(k * CB), CB)
        pltpu.async_copy(x_sh.at[uix_v.at[ksl]], ub, sem)
        pltpu.async_copy(x_sh.at[pix_v.at[ksl]], pb, sem)
        pltpu.async_copy(x_sh.at[nix_v.at[ksl]], nb_, sem).wait()
        pltpu.make_async_copy(xs_hbm.at[c, pl.ds(_i32(0), CB)], ub, sem).wait()
        pltpu.make_async_copy(xs_hbm.at[c, pl.ds(_i32(0), CB)], pb, sem).wait()

        def jbody(j, carry):
            accp, accn = carry
            for u in range(4):
                jv = jnp.full((LN,), j * _i32(4) + _i32(u), _i32)
                uv = plsc.load_gather(ub, [iota, jv])
                pv = plsc.load_gather(pb, [iota, jv])
                nv = plsc.load_gather(nb_, [iota, jv])
                accp = accp + uv * pv
                accn = accn + uv * nv
            return (accp, accn)

        accp, accn = lax.fori_loop(_i32(0), _i32(DH // 4), jbody, (zf, zf))
        pos_o[pl.ds(k * CB, CB)] = accp
        neg_o[pl.ds(k * CB, CB)] = accn

    pltpu.sync_copy(pos_o, posp_hbm.at[c, pl.ds(base, BPW2)])
    pltpu.sync_copy(neg_o, negp_hbm.at[c, pl.ds(base, BPW2)])


def _score_call(xsum, uix, pix, nix):
    return pl.kernel(
        _score_body,
        out_type=(
            jax.ShapeDtypeStruct((NC, B), _f32),
            jax.ShapeDtypeStruct((NC, B), _f32),
        ),
        mesh=_mesh(),
        scratch_types=[
            pltpu.VMEM_SHARED((NNP, DH), _f32),
            pltpu.VMEM((BPW2,), _i32),
            pltpu.VMEM((BPW2,), _i32),
            pltpu.VMEM((BPW2,), _i32),
            pltpu.VMEM((CB, DH), _f32),
            pltpu.VMEM((CB, DH), _f32),
            pltpu.VMEM((CB, DH), _f32),
            pltpu.VMEM((BPW2,), _f32),
            pltpu.VMEM((BPW2,), _f32),
            pltpu.SemaphoreType.DMA,
        ],
        compiler_params=_SC_PARAMS,
    )(xsum, uix, pix, nix)


def _comb_body(pp_ref, np_ref, pos_ref, neg_ref):
    pos_ref[...] = pp_ref[0] + pp_ref[1]
    neg_ref[...] = np_ref[0] + np_ref[1]


def _comb_call(posp, negp):
    return pl.pallas_call(
        _comb_body,
        out_shape=(
            jax.ShapeDtypeStruct((B,), _f32),
            jax.ShapeDtypeStruct((B,), _f32),
        ),
    )(posp, negp)


def kernel(edge_index, user, pos_item, neg_item, user_w, item_w):
    row = edge_index[0].astype(_i32)
    col = edge_index[1].astype(_i32)
    pad = E_PAD - E
    row_p = jnp.concatenate([row, jnp.arange(pad, dtype=_i32) % NN])
    col_p = jnp.concatenate([col, jnp.full((pad,), PAD_COL, _i32)])

    deg2 = _deg_call(col_p)
    dis = _dis_call(deg2).reshape(NPADDED)
    rows, cols, nrm, cnt = _filter_call(row_p, col_p, dis)

    xf = jnp.concatenate(
        [user_w, item_w, jnp.zeros((NNP - NN, D), _f32)], axis=0)

    def _pack_half(xh):
        lo = lax.bitcast_convert_type(
            xh[:, :DHW].astype(jnp.bfloat16), jnp.uint16).astype(jnp.uint32)
        hi = lax.bitcast_convert_type(
            xh[:, DHW:].astype(jnp.bfloat16), jnp.uint16).astype(jnp.uint32)
        return lax.bitcast_convert_type(lo | (hi << 16), _i32)

    x0 = jnp.stack([_pack_half(xf[:, :DH]), _pack_half(xf[:, DH:])], axis=0)
    dz = jnp.zeros((GB2, DH), _f32)
    x1 = _layer_call(x0, rows, cols, nrm, cnt, dz)
    x2 = _layer_call(x1, rows, cols, nrm, cnt, dz)
    x3 = _layer_call(x2, rows, cols, nrm, cnt, dz)
    xsum = _xsum_call(x0, x1, x2, x3)

    uix = user.astype(_i32)
    pix = N_USERS + pos_item.astype(_i32)
    nix = N_USERS + neg_item.astype(_i32)
    posp, negp = _score_call(xsum, uix, pix, nix)
    return _comb_call(posp, negp)


# speedup vs baseline: 3.0668x; 2.9057x over previous
import functools

import jax
import jax.numpy as jnp
from jax import lax
from jax.experimental import pallas as pl
from jax.experimental.pallas import tpu as pltpu
from jax.experimental.pallas import tpu_sc as plsc

N_USERS = 2000
N_ITEMS = 8000
NN = N_USERS + N_ITEMS
D = 256
NLAYER = 3
E = 160000
B = 4096

NC, NS, LN = 2, 16, 16
NW = NC * NS

NPT = 320
NNP = NW * NPT
DEG_R, DEG_C = 80, 128
NPADDED = DEG_R * DEG_C
E_PAD = 163840
EPW = E_PAD // NW
PAD_COL = 10100
CAP = 6144
ECH = 4096
GB = 64
BPW = B // NW
CB = 16

_f32 = jnp.float32
_i32 = jnp.int32

_sc_mesh = None


def _mesh():
    global _sc_mesh
    if _sc_mesh is None:
        _sc_mesh = plsc.VectorSubcoreMesh(
            core_axis_name="c", subcore_axis_name="s",
            num_cores=NC, num_subcores=NS,
        )
    return _sc_mesh


_SC_PARAMS = pltpu.CompilerParams(needs_layout_passes=False)
_SC_PARAMS_NOTC = pltpu.CompilerParams(
    needs_layout_passes=False, use_tc_tiling_on_sc=False)


def _wid():
    return lax.axis_index("c") * NS + lax.axis_index("s")


def _deg_body(col_hbm, out_hbm, deg_v, col_v, rid_v, deg_sh):
    s = lax.axis_index("s")
    c = lax.axis_index("c")
    wid = c * NS + s

    zeros = jnp.zeros((LN,), _f32)
    ones = jnp.ones((LN,), _f32)

    def zbody(r, carry):
        for k in range(DEG_C // LN):
            deg_v[r, pl.ds(k * LN, LN)] = zeros
        return carry

    lax.fori_loop(_i32(0), _i32(DEG_R), zbody, _i32(0))

    base_iota = lax.iota(_i32, LN)
    for i in range(DEG_R // LN):
        rid_v[pl.ds(i * LN, LN)] = base_iota + _i32(i * LN)

    pltpu.sync_copy(col_hbm.at[pl.ds(wid * _i32(EPW), EPW)], col_v)

    def body(i, carry):
        cv = col_v[pl.ds(i * _i32(LN), LN)]
        r = lax.shift_right_logical(cv, _i32(7))
        cc = lax.bitwise_and(cv, _i32(127))
        plsc.addupdate_scatter(deg_v, [r, cc], ones)
        return carry

    lax.fori_loop(_i32(0), _i32(EPW // LN), body, _i32(0))

    @pl.when(s == 0)
    def _():
        pltpu.sync_copy(deg_v, deg_sh)

    plsc.subcore_barrier()

    @pl.when(s != 0)
    def _():
        pltpu.sync_copy(deg_v, deg_sh.at[rid_v], add=True)

    plsc.subcore_barrier()

    @pl.when(s == 0)
    def _():
        pltpu.sync_copy(deg_sh, out_hbm.at[c])


def _deg_call(col_p):
    return pl.kernel(
        _deg_body,
        out_type=jax.ShapeDtypeStruct((NC, DEG_R, DEG_C), _f32),
        mesh=_mesh(),
        scratch_types=[
            pltpu.VMEM((DEG_R, DEG_C), _f32),
            pltpu.VMEM((EPW,), _i32),
            pltpu.VMEM((DEG_R,), _i32),
            pltpu.VMEM_SHARED((DEG_R, DEG_C), _f32),
        ],
        compiler_params=_SC_PARAMS,
    )(col_p)


def _dis_body(deg_ref, dis_ref):
    d = deg_ref[0] + deg_ref[1]
    dis_ref[...] = jnp.where(d > 0, lax.rsqrt(d), 0.0).astype(_f32)


def _dis_call(deg2):
    return pl.pallas_call(
        _dis_body,
        out_shape=jax.ShapeDtypeStruct((DEG_R, DEG_C), _f32),
    )(deg2)


def _filter_body(row_hbm, col_hbm, dis_hbm,
                 rows_hbm, cols_hbm, nrm_hbm, cnt_hbm,
                 dis_v, er_v, ec_v, rows_o, cols_o, nrm_o, cnt_o):
    wid = _wid()
    lo = wid * _i32(NPT)
    lo_v = jnp.full((LN,), lo, _i32)
    hi_v = lo_v + _i32(NPT)

    pltpu.sync_copy(dis_hbm, dis_v)

    iota = lax.iota(_i32, LN)
    zf = jnp.zeros((LN,), _f32)
    zi = jnp.zeros((LN,), _i32)
    onei = jnp.ones((LN,), _i32)

    def ibody(k, carry):
        sl = pl.ds(k * _i32(LN), LN)
        rows_o[sl] = lax.bitwise_and(iota + k * _i32(LN), _i32(8191))
        cols_o[sl] = zi
        nrm_o[sl] = zf
        return carry

    lax.fori_loop(_i32(0), _i32(CAP // LN), ibody, _i32(0))

    def chunk_body(ci, off_v):
        base = ci * _i32(ECH)
        pltpu.sync_copy(row_hbm.at[pl.ds(base, ECH)], er_v)
        pltpu.sync_copy(col_hbm.at[pl.ds(base, ECH)], ec_v)

        def body(i, off_v):
            sl = pl.ds(i * _i32(LN), LN)
            cv = ec_v[sl]
            rv = er_v[sl]
            m = jnp.logical_and(jnp.logical_and(cv >= lo_v, cv < hi_v),
                                cv < jnp.full((LN,), NN, _i32))
            dr = plsc.load_gather(dis_v, [rv])
            dc = plsc.load_gather(dis_v, [cv])
            nv = dr * dc
            mi = jnp.where(m, onei, zi)
            pc = plsc.cumsum(mi)
            idx = off_v + pc - onei
            plsc.store_scatter(rows_o, [idx], rv, mask=m)
            plsc.store_scatter(cols_o, [idx], cv - lo_v, mask=m)
            plsc.store_scatter(nrm_o, [idx], nv, mask=m)
            cnt = plsc.all_reduce_population_count(m)
            if cnt.shape != (LN,):
                cnt = jnp.broadcast_to(cnt, (LN,)).astype(_i32)
            return off_v + cnt

        return lax.fori_loop(_i32(0), _i32(ECH // LN), body, off_v)

    off_v = lax.fori_loop(_i32(0), _i32(E_PAD // ECH), chunk_body, zi)
    cnt_o[...] = off_v

    pltpu.sync_copy(rows_o, rows_hbm.at[wid, _i32(0)])
    pltpu.sync_copy(cols_o, cols_hbm.at[wid, _i32(0)])
    pltpu.sync_copy(nrm_o, nrm_hbm.at[wid, _i32(0)])
    pltpu.sync_copy(cnt_o, cnt_hbm.at[wid, _i32(0)])


def _filter_call(row_p, col_p, dis_flat):
    return pl.kernel(
        _filter_body,
        out_type=(
            jax.ShapeDtypeStruct((NW, 1, CAP), _i32),
            jax.ShapeDtypeStruct((NW, 1, CAP), _i32),
            jax.ShapeDtypeStruct((NW, 1, CAP), _f32),
            jax.ShapeDtypeStruct((NW, 1, LN), _i32),
        ),
        mesh=_mesh(),
        scratch_types=[
            pltpu.VMEM((NPADDED,), _f32),
            pltpu.VMEM((ECH,), _i32),
            pltpu.VMEM((ECH,), _i32),
            pltpu.VMEM((CAP,), _i32),
            pltpu.VMEM((CAP,), _i32),
            pltpu.VMEM((CAP,), _f32),
            pltpu.VMEM((LN,), _i32),
        ],
        compiler_params=_SC_PARAMS,
    )(row_p, col_p, dis_flat)


DH = 128
DHW = DH // 2
GB2 = 64
NUNIT = 2
WCH = 32


def _gather_batch(x_sh, rows_v, base, gb, sem):
    return pltpu.async_copy(x_sh.at[rows_v.at[pl.ds(base, GB2)]], gb, sem)


def _gather_batch_hbm(x_hbm, c, rows_v, base, gb, sem):
    return pltpu.async_copy(
        x_hbm.at[c].at[rows_v.at[pl.ds(base, GB2)]], gb, sem)


def _drain(x_hbm, c, gb, sem):
    pltpu.make_async_copy(x_hbm.at[c, pl.ds(_i32(0), GB2)], gb, sem).wait()


def _edge_compute(gb, msg, nrm_v, base, iota, onei):
    def ebody(e, carry):
        nsplat = plsc.load_gather(nrm_v, [jnp.full((LN,), base + e, _i32)])
        for k in range(DHW // LN):
            w = gb[e, pl.ds(k * LN, LN)]
            lo = plsc.bitcast(lax.shift_left(w, _i32(16)), _f32)
            hi = plsc.bitcast(lax.bitwise_and(w, _i32(-65536)), _f32)
            msg[e, pl.ds(k * LN, LN)] = lo * nsplat
            msg[e, pl.ds(DHW + k * LN, LN)] = hi * nsplat
        return carry

    lax.fori_loop(_i32(0), _i32(GB2), ebody, _i32(0))


def _layer_body(x_hbm, rows_hbm, cols_hbm, nrm_hbm, cnt_hbm, dz_hbm, y_hbm,
                x_sh, acc_sh, gb0, gb1, gb2, msg, accbuf, ybuf,
                rows_v, cols_v, nrm_v, oix, cnt_v,
                sem0, sem1, sem2, ssem):
    c = lax.axis_index("c")
    s = lax.axis_index("s")
    stripe = NNP // NS
    st = s * _i32(stripe)
    pltpu.sync_copy(x_hbm.at[c].at[pl.ds(st, stripe)],
                    x_sh.at[pl.ds(st, stripe)])
    iota = lax.iota(_i32, LN)
    zeros = jnp.zeros((LN,), _f32)
    onei = jnp.ones((LN,), _i32)
    soff = s * _i32(NPT)
    plsc.subcore_barrier()

    gbs = (gb0, gb1, gb2)
    sems = (sem0, sem1, sem2)

    for ui in range(NUNIT):
        u = s + _i32(ui * NS)
        pltpu.sync_copy(rows_hbm.at[u, _i32(0)], rows_v)
        pltpu.sync_copy(cols_hbm.at[u, _i32(0)], cols_v)
        pltpu.sync_copy(nrm_hbm.at[u, _i32(0)], nrm_v)
        pltpu.sync_copy(cnt_hbm.at[u, _i32(0)], cnt_v)

        def abody(r, carry):
            accbuf[r, pl.ds(0, LN)] = zeros
            for k in range(1, DH // LN):
                accbuf[r, pl.ds(k * LN, LN)] = zeros
            return carry

        lax.fori_loop(_i32(0), _i32(WCH), abody, _i32(0))
        for k in range(NPT // WCH):
            pltpu.sync_copy(
                accbuf, acc_sh.at[pl.ds(soff + _i32(k * WCH), WCH)])

        n = jnp.max(cnt_v[...])
        ntr = lax.div(n + _i32(3 * GB2 - 1), _i32(3 * GB2))

        for q in range(3):
            _gather_batch(x_sh, rows_v, _i32(q * GB2), gbs[q], sems[q])

        def pbody(p, carry):
            for q in range(3):
                b = p * _i32(3) + _i32(q)
                bb = b * _i32(GB2)
                @pl.when(jnp.logical_or(p > 0, _i32(q) > 0))
                def _():
                    pltpu.make_async_copy(dz_hbm, msg, ssem).wait()

                _drain(x_hbm, c, gbs[q], sems[q])
                _edge_compute(gbs[q], msg, nrm_v, bb, iota, onei)
                for g in range(GB2 // LN):
                    oix[pl.ds(g * LN, LN)] = (
                        cols_v[pl.ds(bb + _i32(g * LN), LN)] + soff)
                pltpu.async_copy(msg, acc_sh.at[oix], ssem, add=True)
                _gather_batch(x_sh, rows_v, bb + _i32(3 * GB2), gbs[q],
                              sems[q])
            return carry

        lax.fori_loop(_i32(0), ntr, pbody, _i32(0))
        for q in range(3):
            _drain(x_hbm, c, gbs[q], sems[q])
        pltpu.make_async_copy(dz_hbm, msg, ssem).wait()

        def wbody(rb, carry):
            pltpu.sync_copy(
                acc_sh.at[pl.ds(soff + rb * _i32(WCH), WCH)], accbuf)

            def rbody(rr, carry2):
                for k in range(DHW // LN):
                    a = accbuf[rr, pl.ds(k * LN, LN)]
                    b = accbuf[rr, pl.ds(DHW + k * LN, LN)]
                    pk = plsc.pack(a, b, format=plsc.PackFormat.INTERLEAVED)
                    ybuf[rr, pl.ds(k * LN, LN)] = plsc.bitcast(pk, _i32)
                return carry2

            lax.fori_loop(_i32(0), _i32(WCH), rbody, _i32(0))
            pltpu.sync_copy(
                ybuf, y_hbm.at[c, pl.ds(u * _i32(NPT) + rb * _i32(WCH), WCH)])
            return carry

        lax.fori_loop(_i32(0), _i32(NPT // WCH), wbody, _i32(0))


def _layer_call(x, rows, cols, nrm, cnt, dz):
    return pl.kernel(
        _layer_body,
        out_type=jax.ShapeDtypeStruct((NC, NNP, DHW), _i32),
        mesh=_mesh(),
        scratch_types=[
            pltpu.VMEM_SHARED((NNP, DHW), _i32),
            pltpu.VMEM_SHARED((NS * NPT, DH), _f32),
            pltpu.VMEM((GB2, DHW), _i32),
            pltpu.VMEM((GB2, DHW), _i32),
            pltpu.VMEM((GB2, DHW), _i32),
            pltpu.VMEM((GB2, DH), _f32),
            pltpu.VMEM((WCH, DH), _f32),
            pltpu.VMEM((WCH, DHW), _i32),
            pltpu.VMEM((CAP,), _i32),
            pltpu.VMEM((CAP,), _i32),
            pltpu.VMEM((CAP,), _f32),
            pltpu.VMEM((GB2,), _i32),
            pltpu.VMEM((LN,), _i32),
            pltpu.SemaphoreType.DMA,
            pltpu.SemaphoreType.DMA,
            pltpu.SemaphoreType.DMA,
            pltpu.SemaphoreType.DMA,
        ],
        compiler_params=_SC_PARAMS_NOTC,
    )(x, rows, cols, nrm, cnt, dz)


def _unpack_tc(w):
    lo = lax.bitcast_convert_type(lax.shift_left(w, _i32(16)), _f32)
    hi = lax.bitcast_convert_type(lax.bitwise_and(w, _i32(-65536)), _f32)
    return lo, hi


def _xsum_body(a_ref, b_ref, c_ref, d_ref, o_ref):
    alo, ahi = _unpack_tc(a_ref[...])
    blo, bhi = _unpack_tc(b_ref[...])
    clo, chi = _unpack_tc(c_ref[...])
    dlo, dhi = _unpack_tc(d_ref[...])
    o_ref[:, :, :DHW] = 0.25 * (alo + blo + clo + dlo)
    o_ref[:, :, DHW:] = 0.25 * (ahi + bhi + chi + dhi)


def _xsum_call(x0, x1, x2, x3):
    iblk = pl.BlockSpec((1, 2048, DHW), lambda i, j: (i, j, _i32(0)))
    oblk = pl.BlockSpec((1, 2048, DH), lambda i, j: (i, j, _i32(0)))
    return pl.pallas_call(
        _xsum_body,
        out_shape=jax.ShapeDtypeStruct((NC, NNP, DH), _f32),
        grid=(NC, NNP // 2048),
        in_specs=[iblk, iblk, iblk, iblk],
        out_specs=oblk,
    )(x0, x1, x2, x3)


BPW2 = B // NS


def _score_body(xs_hbm, uix_hbm, pix_hbm, nix_hbm,
                posp_hbm, negp_hbm,
                x_sh, uix_v, pix_v, nix_v, ub, pb, nb_, pos_o, neg_o, sem):
    c = lax.axis_index("c")
    s = lax.axis_index("s")

    @pl.when(s == 0)
    def _():
        pltpu.sync_copy(xs_hbm.at[c], x_sh)

    base = s * _i32(BPW2)
    pltpu.sync_copy(uix_hbm.at[pl.ds(base, BPW2)], uix_v)
    pltpu.sync_copy(pix_hbm.at[pl.ds(base, BPW2)], pix_v)
    pltpu.sync_copy(nix_hbm.at[pl.ds(base, BPW2)], nix_v)
    plsc.subcore_barrier()

    iota = lax.iota(_i32, LN)
    zf = jnp.zeros((LN,), _f32)

    for k in range(BPW2 // CB):
        ksl = pl.ds(_i32(k * CB), CB)
        pltpu.async_copy(x_sh.at[uix_v.at[ksl]], ub, sem)
        pltpu.async_copy(x_sh.at[pix_v.at[ksl]], pb, sem)
        pltpu.async_copy(x_sh.at[nix_v.at[ksl]], nb_, sem).wait()
        pltpu.make_async_copy(xs_hbm.at[c, pl.ds(_i32(0), CB)], ub, sem).wait()
        pltpu.make_async_copy(xs_hbm.at[c, pl.ds(_i32(0), CB)], pb, sem).wait()

        def jbody(j, carry):
            accp, accn = carry
            for u in range(4):
                jv = jnp.full((LN,), j * _i32(4) + _i32(u), _i32)
                uv = plsc.load_gather(ub, [iota, jv])
                pv = plsc.load_gather(pb, [iota, jv])
                nv = plsc.load_gather(nb_, [iota, jv])
                accp = accp + uv * pv
                accn = accn + uv * nv
            return (accp, accn)

        accp, accn = lax.fori_loop(_i32(0), _i32(DH // 4), jbody, (zf, zf))
        pos_o[pl.ds(k * CB, CB)] = accp
        neg_o[pl.ds(k * CB, CB)] = accn

    pltpu.sync_copy(pos_o, posp_hbm.at[c, pl.ds(base, BPW2)])
    pltpu.sync_copy(neg_o, negp_hbm.at[c, pl.ds(base, BPW2)])


def _score_call(xsum, uix, pix, nix):
    return pl.kernel(
        _score_body,
        out_type=(
            jax.ShapeDtypeStruct((NC, B), _f32),
            jax.ShapeDtypeStruct((NC, B), _f32),
        ),
        mesh=_mesh(),
        scratch_types=[
            pltpu.VMEM_SHARED((NNP, DH), _f32),
            pltpu.VMEM((BPW2,), _i32),
            pltpu.VMEM((BPW2,), _i32),
            pltpu.VMEM((BPW2,), _i32),
            pltpu.VMEM((CB, DH), _f32),
            pltpu.VMEM((CB, DH), _f32),
            pltpu.VMEM((CB, DH), _f32),
            pltpu.VMEM((BPW2,), _f32),
            pltpu.VMEM((BPW2,), _f32),
            pltpu.SemaphoreType.DMA,
        ],
        compiler_params=_SC_PARAMS,
    )(xsum, uix, pix, nix)


def _comb_body(pp_ref, np_ref, pos_ref, neg_ref):
    pos_ref[...] = pp_ref[0] + pp_ref[1]
    neg_ref[...] = np_ref[0] + np_ref[1]


def _comb_call(posp, negp):
    return pl.pallas_call(
        _comb_body,
        out_shape=(
            jax.ShapeDtypeStruct((B,), _f32),
            jax.ShapeDtypeStruct((B,), _f32),
        ),
    )(posp, negp)


def kernel(edge_index, user, pos_item, neg_item, user_w, item_w):
    row = edge_index[0].astype(_i32)
    col = edge_index[1].astype(_i32)
    pad = E_PAD - E
    row_p = jnp.concatenate([row, jnp.arange(pad, dtype=_i32) % NN])
    col_p = jnp.concatenate([col, jnp.full((pad,), PAD_COL, _i32)])

    deg2 = _deg_call(col_p)
    dis = _dis_call(deg2).reshape(NPADDED)
    rows, cols, nrm, cnt = _filter_call(row_p, col_p, dis)

    xf = jnp.concatenate(
        [user_w, item_w, jnp.zeros((NNP - NN, D), _f32)], axis=0)

    def _pack_half(xh):
        lo = lax.bitcast_convert_type(
            xh[:, :DHW].astype(jnp.bfloat16), jnp.uint16).astype(jnp.uint32)
        hi = lax.bitcast_convert_type(
            xh[:, DHW:].astype(jnp.bfloat16), jnp.uint16).astype(jnp.uint32)
        return lax.bitcast_convert_type(lo | (hi << 16), _i32)

    x0 = jnp.stack([_pack_half(xf[:, :DH]), _pack_half(xf[:, DH:])], axis=0)
    dz = jnp.zeros((GB2, DH), _f32)
    x1 = _layer_call(x0, rows, cols, nrm, cnt, dz)
    x2 = _layer_call(x1, rows, cols, nrm, cnt, dz)
    x3 = _layer_call(x2, rows, cols, nrm, cnt, dz)
    xsum = _xsum_call(x0, x1, x2, x3)

    uix = user.astype(_i32)
    pix = N_USERS + pos_item.astype(_i32)
    nix = N_USERS + neg_item.astype(_i32)
    posp, negp = _score_call(xsum, uix, pix, nix)
    return _comb_call(posp, negp)
